# baseline (speedup 1.0000x reference)
import functools
import jax
import jax.numpy as jnp
from jax import lax
from jax.experimental import pallas as pl
from jax.experimental.pallas import tpu as pltpu
from jax.experimental.pallas import tpu_sc as plsc

N = 10000
E = 320000
H = 128
L = 6
NUM_GRAPHS = 16
NUM_CLINICAL = 8
NUM_CLASSES = 2
EPS = 1e-7
BN_EPS = 1e-5

BLK = 512
NBLK = E // BLK
ACCW = N * 8


def _sc_edge_kernel(xg, srcE, dstE, attr, Wsp, tvec, out,
                    srcb, dstb, attrb, idxb, xw, wsbuf, tbuf, acc, sem):
    c = lax.axis_index("c")
    s = lax.axis_index("s")
    g = c * 16 + s
    grp = g // 4
    wch0 = (g % 4) * 4

    pltpu.sync_copy(Wsp.at[pl.ds(g * 256, 256)], wsbuf)
    pltpu.sync_copy(tvec, tbuf)
    tv = tbuf[...]
    wsp = [[wsbuf[pl.ds((k * 4 + ch) * 16, 16)] for ch in range(4)]
           for k in range(4)]

    zero16 = jnp.zeros((16,), jnp.float32)
    iota16 = lax.iota(jnp.int32, 16)
    attr_base = [iota16 * 4 + k for k in range(4)]
    row_base = iota16
    col_spl = [jnp.zeros((16,), jnp.int32) + (wch0 + ch) for ch in range(4)]

    def zero_body(i, _):
        acc[pl.ds(i * 16, 16)] = zero16
        return 0

    lax.fori_loop(0, ACCW // 16, zero_body, 0)

    def blk_body(blk, _):
        off = blk * BLK
        pltpu.sync_copy(srcE.at[pl.ds(off, BLK)], srcb)
        pltpu.sync_copy(dstE.at[pl.ds(off, BLK)], dstb)
        pltpu.sync_copy(attr.at[pl.ds(off * 4, BLK * 4)], attrb)

        def idx_body(i, _):
            s16 = srcb[pl.ds(i * 16, 16)]
            idxb[pl.ds(i * 16, 16)] = s16 * 8 + grp
            return 0

        lax.fori_loop(0, BLK // 16, idx_body, 0)
        pltpu.async_copy(xg.at[idxb], xw, sem).wait()

        def grp_body(i, _):
            d16 = dstb[pl.ds(i * 16, 16)]
            av = [plsc.load_gather(attrb, [attr_base[k] + i * 64])
                  for k in range(4)]
            rows = row_base + i * 16
            nbase = d16 * 8
            for ch in range(4):
                xcol = plsc.load_gather(xw, [rows, col_spl[ch]])
                ea = (av[0] * wsp[0][ch] + av[1] * wsp[1][ch]
                      + av[2] * wsp[2][ch] + av[3] * wsp[3][ch])
                msg = jnp.maximum(xcol + ea, 0.0) + EPS
                pv = jnp.exp(msg * tv)
                plsc.addupdate_scatter(acc, [nbase + ch], msg * pv)
                plsc.addupdate_scatter(acc, [nbase + (ch + 4)], pv)
            return 0

        lax.fori_loop(0, BLK // 16, grp_body, 0)
        return 0

    lax.fori_loop(0, NBLK, blk_body, 0)

    pltpu.sync_copy(acc, out.at[pl.ds(g * ACCW, ACCW)])


@functools.partial(
    pl.kernel,
    mesh=plsc.VectorSubcoreMesh(core_axis_name="c", subcore_axis_name="s"),
    out_type=jax.ShapeDtypeStruct((32 * ACCW,), jnp.float32),
    scratch_types=[
        pltpu.VMEM((BLK,), jnp.int32),
        pltpu.VMEM((BLK,), jnp.int32),
        pltpu.VMEM((BLK * 4,), jnp.float32),
        pltpu.VMEM((BLK,), jnp.int32),
        pltpu.VMEM((BLK, 16), jnp.float32),
        pltpu.VMEM((256,), jnp.float32),
        pltpu.VMEM((16,), jnp.float32),
        pltpu.VMEM((ACCW,), jnp.float32),
        pltpu.SemaphoreType.DMA,
    ],
    compiler_params=pltpu.CompilerParams(use_tc_tiling_on_sc=False,
                                         needs_layout_passes=False),
)
def _sc_edge_pass(xg, srcE, dstE, attr, Wsp, tvec, out, *scratch):
    _sc_edge_kernel(xg, srcE, dstE, attr, Wsp, tvec, out, *scratch)


ROWB = 1000


def _tc_layer_kernel(nu_ref, de_ref, v_ref, h_ref, W1_ref, g1_ref, b1_ref,
                     W2_ref, gnx_ref, bnx_ref, hout_ref, vout_ref):
    m = nu_ref[...] / (de_ref[...] + 1e-16) + v_ref[...]
    t1 = jnp.dot(m, W1_ref[...], preferred_element_type=jnp.float32)
    t1 = g1_ref[...] * (t1 / jnp.sqrt(1.0 + BN_EPS)) + b1_ref[...]
    t1 = jnp.maximum(t1, 0.0)
    z = jnp.dot(t1, W2_ref[...], preferred_element_type=jnp.float32)
    h = h_ref[...] + z
    hout_ref[...] = h
    vout_ref[...] = jnp.maximum(
        gnx_ref[...] * (h / jnp.sqrt(1.0 + BN_EPS)) + bnx_ref[...], 0.0)


def _tc_layer(numer, denom, v_in, h_prev, W1, g1, b1, W2, gn_next, bn_next):
    grid = (N // ROWB,)
    rb = lambda i: (i, 0)
    cons = lambda i: (0, 0)
    return pl.pallas_call(
        _tc_layer_kernel,
        grid=grid,
        in_specs=[
            pl.BlockSpec((ROWB, H), rb),
            pl.BlockSpec((ROWB, H), rb),
            pl.BlockSpec((ROWB, H), rb),
            pl.BlockSpec((ROWB, H), rb),
            pl.BlockSpec((H, 2 * H), cons),
            pl.BlockSpec((1, 2 * H), cons),
            pl.BlockSpec((1, 2 * H), cons),
            pl.BlockSpec((2 * H, H), cons),
            pl.BlockSpec((1, H), cons),
            pl.BlockSpec((1, H), cons),
        ],
        out_specs=[
            pl.BlockSpec((ROWB, H), rb),
            pl.BlockSpec((ROWB, H), rb),
        ],
        out_shape=[
            jax.ShapeDtypeStruct((N, H), jnp.float32),
            jax.ShapeDtypeStruct((N, H), jnp.float32),
        ],
    )(numer, denom, v_in, h_prev, W1, g1.reshape(1, 2 * H),
      b1.reshape(1, 2 * H), W2, gn_next.reshape(1, H), bn_next.reshape(1, H))


def _final_kernel(f_ref, batch_ref, clin_ref, wc_ref, bc_ref, out_ref):
    f = f_ref[...]
    batch = batch_ref[...]
    gids = jax.lax.broadcasted_iota(jnp.int32, (NUM_GRAPHS, N), 0)
    mask = (gids == batch).astype(jnp.float32)
    sums = jnp.dot(mask, f, preferred_element_type=jnp.float32)
    cnt = jnp.sum(mask, axis=1, keepdims=True)
    pooled = sums / jnp.maximum(cnt, 1.0)
    wc = wc_ref[...]
    out_ref[...] = (
        jnp.dot(pooled, wc[:H, :], preferred_element_type=jnp.float32)
        + jnp.dot(clin_ref[...], wc[H:, :], preferred_element_type=jnp.float32)
        + bc_ref[...])


def _final_stage(f, batch, clinical, W_cls, b_cls):
    return pl.pallas_call(
        _final_kernel,
        out_shape=jax.ShapeDtypeStruct((NUM_GRAPHS, NUM_CLASSES), jnp.float32),
    )(f, batch.reshape(1, N).astype(jnp.int32), clinical, W_cls,
      b_cls.reshape(1, NUM_CLASSES))


@jax.jit
def _run(x, edge_index, edge_attr, batch, clinical, W_edge, t, W1, g1, b1,
         W2, gn, bn, W_cls, b_cls):
    srcE = edge_index[0].astype(jnp.int32)
    dstE = edge_index[1].astype(jnp.int32)
    attr_flat = edge_attr.reshape(E * 4)
    v_in = x
    h = jnp.zeros((N, H), jnp.float32)
    for l in range(L):
        tvec = jnp.full((16,), t[l], jnp.float32)
        Wsp = jnp.broadcast_to(
            W_edge[l].reshape(4, 32, 4).transpose(1, 0, 2)[..., None],
            (32, 4, 4, 16)).reshape(32 * 256)
        xg = v_in.reshape(N * 8, 16)
        outf = _sc_edge_pass(xg, srcE, dstE, attr_flat, Wsp, tvec)
        r = outf.reshape(32, N, 8)
        numer = r[:, :, :4].transpose(1, 0, 2).reshape(N, H)
        denom = r[:, :, 4:].transpose(1, 0, 2).reshape(N, H)
        gi = (l + 1) if l + 1 < L else 0
        h, v_in = _tc_layer(numer, denom, v_in, h, W1[l], g1[l], b1[l],
                            W2[l], gn[gi], bn[gi])
    return _final_stage(v_in, batch, clinical, W_cls, b_cls)


def kernel(x, edge_index, edge_attr, batch, clinical, W_edge, t, W1, g1, b1,
           W2, gn, bn, W_cls, b_cls):
    return _run(x, edge_index, edge_attr, batch, clinical, W_edge, t, W1, g1,
                b1, W2, gn, bn, W_cls, b_cls)


# speedup vs baseline: 1.4801x; 1.4801x over previous
import functools
import jax
import jax.numpy as jnp
from jax import lax
from jax.experimental import pallas as pl
from jax.experimental.pallas import tpu as pltpu
from jax.experimental.pallas import tpu_sc as plsc

N = 10000
E = 320000
H = 128
L = 6
NUM_GRAPHS = 16
NUM_CLINICAL = 8
NUM_CLASSES = 2
EPS = 1e-7
BN_EPS = 1e-5

BLK = 800
NBLK = E // BLK
ACCW = N * 8


def _sc_edge_kernel(xg, srcE, dstE, attr, Wsp, tvec, out,
                    srcb, dstb, attrb, idxb, xw, wsbuf, tbuf, acc, sems):
    c = lax.axis_index("c")
    s = lax.axis_index("s")
    g = c * 16 + s
    grp = g // 4
    wch0 = (g % 4) * 4

    pltpu.sync_copy(Wsp.at[pl.ds(g * 256, 256)], wsbuf)
    pltpu.sync_copy(tvec, tbuf)
    tv = tbuf[...]
    wsp = [[wsbuf[pl.ds((k * 4 + ch) * 16, 16)] for ch in range(4)]
           for k in range(4)]

    zero16 = jnp.zeros((16,), jnp.float32)
    iota16 = lax.iota(jnp.int32, 16)
    attr_base = [iota16 * 4 + k for k in range(4)]
    row_base = iota16
    col_spl = [jnp.zeros((16,), jnp.int32) + (wch0 + ch) for ch in range(4)]

    def zero_body(i, _):
        acc[pl.ds(i * 16, 16)] = zero16
        return 0

    lax.fori_loop(0, ACCW // 16, zero_body, 0)

    def slab_copies(j, p):
        off = j * BLK
        return [
            pltpu.make_async_copy(srcE.at[pl.ds(off, BLK)], srcb.at[p],
                                  sems.at[p, 0]),
            pltpu.make_async_copy(dstE.at[pl.ds(off, BLK)], dstb.at[p],
                                  sems.at[p, 1]),
            pltpu.make_async_copy(attr.at[pl.ds(off * 4, BLK * 4)],
                                  attrb.at[pl.ds(p * BLK * 4, BLK * 4)],
                                  sems.at[p, 2]),
        ]

    def gather_copy(p):
        return pltpu.make_async_copy(xg.at[idxb.at[p]],
                                     xw.at[pl.ds(p * BLK, BLK)],
                                     sems.at[p, 3])

    def start_slabs(j, p):
        for cp in slab_copies(j, p):
            cp.start()

    def prep_gather(j, p):
        for cp in slab_copies(j, p):
            cp.wait()

        def idx_body(i, _):
            s16 = srcb[p, pl.ds(i * 16, 16)]
            idxb[p, pl.ds(i * 16, 16)] = s16 * 8 + grp
            return 0

        lax.fori_loop(0, BLK // 16, idx_body, 0)
        gather_copy(p).start()

    def compute(p):
        gather_copy(p).wait()

        def grp_body(i, _):
            d16 = dstb[p, pl.ds(i * 16, 16)]
            av = [plsc.load_gather(attrb, [attr_base[k] + (p * BLK * 4
                                                           + i * 64)])
                  for k in range(4)]
            rows = row_base + (p * BLK + i * 16)
            nbase = d16 * 8
            for ch in range(4):
                xcol = plsc.load_gather(xw, [rows, col_spl[ch]])
                ea = (av[0] * wsp[0][ch] + av[1] * wsp[1][ch]
                      + av[2] * wsp[2][ch] + av[3] * wsp[3][ch])
                msg = jnp.maximum(xcol + ea, 0.0) + EPS
                pv = jnp.exp(msg * tv)
                plsc.addupdate_scatter(acc, [nbase + ch], msg * pv)
                plsc.addupdate_scatter(acc, [nbase + (ch + 4)], pv)
            return 0

        lax.fori_loop(0, BLK // 16, grp_body, 0)

    start_slabs(0, 0)
    prep_gather(0, 0)
    start_slabs(1, 1)

    def pair_body(i, _):
        for b in range(2):
            j = i * 2 + b
            p = b
            q = 1 - b

            @pl.when(j + 1 < NBLK)
            def _():
                prep_gather(j + 1, q)

            compute(p)

            @pl.when(j + 2 < NBLK)
            def _():
                start_slabs(j + 2, p)

        return 0

    lax.fori_loop(0, NBLK // 2, pair_body, 0)

    pltpu.sync_copy(acc, out.at[pl.ds(g * ACCW, ACCW)])


@functools.partial(
    pl.kernel,
    mesh=plsc.VectorSubcoreMesh(core_axis_name="c", subcore_axis_name="s"),
    out_type=jax.ShapeDtypeStruct((32 * ACCW,), jnp.float32),
    scratch_types=[
        pltpu.VMEM((2, BLK), jnp.int32),
        pltpu.VMEM((2, BLK), jnp.int32),
        pltpu.VMEM((2 * BLK * 4,), jnp.float32),
        pltpu.VMEM((2, BLK), jnp.int32),
        pltpu.VMEM((2 * BLK, 16), jnp.float32),
        pltpu.VMEM((256,), jnp.float32),
        pltpu.VMEM((16,), jnp.float32),
        pltpu.VMEM((ACCW,), jnp.float32),
        pltpu.SemaphoreType.DMA((2, 4)),
    ],
    compiler_params=pltpu.CompilerParams(use_tc_tiling_on_sc=False,
                                         needs_layout_passes=False),
)
def _sc_edge_pass(xg, srcE, dstE, attr, Wsp, tvec, out, *scratch):
    _sc_edge_kernel(xg, srcE, dstE, attr, Wsp, tvec, out, *scratch)


ROWB = 1000


def _tc_layer_kernel(nu_ref, de_ref, v_ref, h_ref, W1_ref, g1_ref, b1_ref,
                     W2_ref, gnx_ref, bnx_ref, hout_ref, vout_ref):
    m = nu_ref[...] / (de_ref[...] + 1e-16) + v_ref[...]
    t1 = jnp.dot(m, W1_ref[...], preferred_element_type=jnp.float32)
    t1 = g1_ref[...] * (t1 / jnp.sqrt(1.0 + BN_EPS)) + b1_ref[...]
    t1 = jnp.maximum(t1, 0.0)
    z = jnp.dot(t1, W2_ref[...], preferred_element_type=jnp.float32)
    h = h_ref[...] + z
    hout_ref[...] = h
    vout_ref[...] = jnp.maximum(
        gnx_ref[...] * (h / jnp.sqrt(1.0 + BN_EPS)) + bnx_ref[...], 0.0)


def _tc_layer(numer, denom, v_in, h_prev, W1, g1, b1, W2, gn_next, bn_next):
    grid = (N // ROWB,)
    rb = lambda i: (i, 0)
    cons = lambda i: (0, 0)
    return pl.pallas_call(
        _tc_layer_kernel,
        grid=grid,
        in_specs=[
            pl.BlockSpec((ROWB, H), rb),
            pl.BlockSpec((ROWB, H), rb),
            pl.BlockSpec((ROWB, H), rb),
            pl.BlockSpec((ROWB, H), rb),
            pl.BlockSpec((H, 2 * H), cons),
            pl.BlockSpec((1, 2 * H), cons),
            pl.BlockSpec((1, 2 * H), cons),
            pl.BlockSpec((2 * H, H), cons),
            pl.BlockSpec((1, H), cons),
            pl.BlockSpec((1, H), cons),
        ],
        out_specs=[
            pl.BlockSpec((ROWB, H), rb),
            pl.BlockSpec((ROWB, H), rb),
        ],
        out_shape=[
            jax.ShapeDtypeStruct((N, H), jnp.float32),
            jax.ShapeDtypeStruct((N, H), jnp.float32),
        ],
    )(numer, denom, v_in, h_prev, W1, g1.reshape(1, 2 * H),
      b1.reshape(1, 2 * H), W2, gn_next.reshape(1, H), bn_next.reshape(1, H))


def _final_kernel(f_ref, batch_ref, clin_ref, wc_ref, bc_ref, out_ref):
    f = f_ref[...]
    batch = batch_ref[...]
    gids = jax.lax.broadcasted_iota(jnp.int32, (NUM_GRAPHS, N), 0)
    mask = (gids == batch).astype(jnp.float32)
    sums = jnp.dot(mask, f, preferred_element_type=jnp.float32)
    cnt = jnp.sum(mask, axis=1, keepdims=True)
    pooled = sums / jnp.maximum(cnt, 1.0)
    wc = wc_ref[...]
    out_ref[...] = (
        jnp.dot(pooled, wc[:H, :], preferred_element_type=jnp.float32)
        + jnp.dot(clin_ref[...], wc[H:, :], preferred_element_type=jnp.float32)
        + bc_ref[...])


def _final_stage(f, batch, clinical, W_cls, b_cls):
    return pl.pallas_call(
        _final_kernel,
        out_shape=jax.ShapeDtypeStruct((NUM_GRAPHS, NUM_CLASSES), jnp.float32),
    )(f, batch.reshape(1, N).astype(jnp.int32), clinical, W_cls,
      b_cls.reshape(1, NUM_CLASSES))


@jax.jit
def _run(x, edge_index, edge_attr, batch, clinical, W_edge, t, W1, g1, b1,
         W2, gn, bn, W_cls, b_cls):
    srcE = edge_index[0].astype(jnp.int32)
    dstE = edge_index[1].astype(jnp.int32)
    attr_flat = edge_attr.reshape(E * 4)
    v_in = x
    h = jnp.zeros((N, H), jnp.float32)
    for l in range(L):
        tvec = jnp.full((16,), t[l], jnp.float32)
        Wsp = jnp.broadcast_to(
            W_edge[l].reshape(4, 32, 4).transpose(1, 0, 2)[..., None],
            (32, 4, 4, 16)).reshape(32 * 256)
        xg = v_in.reshape(N * 8, 16)
        outf = _sc_edge_pass(xg, srcE, dstE, attr_flat, Wsp, tvec)
        r = outf.reshape(32, N, 8)
        numer = r[:, :, :4].transpose(1, 0, 2).reshape(N, H)
        denom = r[:, :, 4:].transpose(1, 0, 2).reshape(N, H)
        gi = (l + 1) if l + 1 < L else 0
        h, v_in = _tc_layer(numer, denom, v_in, h, W1[l], g1[l], b1[l],
                            W2[l], gn[gi], bn[gi])
    return _final_stage(v_in, batch, clinical, W_cls, b_cls)


def kernel(x, edge_index, edge_attr, batch, clinical, W_edge, t, W1, g1, b1,
           W2, gn, bn, W_cls, b_cls):
    return _run(x, edge_index, edge_attr, batch, clinical, W_edge, t, W1, g1,
                b1, W2, gn, bn, W_cls, b_cls)
